# baseline (speedup 1.0000x reference)
import functools

import jax
import jax.numpy as jnp
from jax import lax
from jax.experimental import pallas as pl
from jax.experimental.pallas import tpu as pltpu
from jax.experimental.pallas import tpu_sc as plsc

_B, _C, _H, _W = 2, 128, 32, 32
_HW = _H * _W
_NQ = 25
_A = float(_NQ - 1)
_K_COEF = 0.5
_NTILES = 32
_RPT = (_B * _HW) // _NTILES
_NB = 32
_CHUNK = 256


def _scores_kernel(q_ref, imgf_ref, grid_ref, out_ref):
    gx = grid_ref[0, :, 0:1]
    gy = grid_ref[0, :, 1:2]
    x = (gx + 1.0) * (_W / 2.0) - 0.5
    y = (gy + 1.0) * (_H / 2.0) - 0.5
    x0 = jnp.floor(x)
    y0 = jnp.floor(y)
    x1 = x0 + 1.0
    y1 = y0 + 1.0
    wx1 = x - x0
    wx0 = 1.0 - wx1
    wy1 = y - y0
    wy0 = 1.0 - wy1

    iota_p = jax.lax.broadcasted_iota(jnp.int32, (_HW, _HW), 1)

    def tap_mat(xi, yi, wgt):
        valid = (xi >= 0.0) & (xi <= _W - 1.0) & (yi >= 0.0) & (yi <= _H - 1.0)
        w = jnp.where(valid, wgt, 0.0)
        xc = jnp.clip(xi, 0.0, _W - 1.0).astype(jnp.int32)
        yc = jnp.clip(yi, 0.0, _H - 1.0).astype(jnp.int32)
        idx = yc * _W + xc
        return jnp.where(iota_p == idx, w, 0.0)

    g_mat = (tap_mat(x0, y0, wx0 * wy0) + tap_mat(x1, y0, wx1 * wy0)
             + tap_mat(x0, y1, wx0 * wy1) + tap_mat(x1, y1, wx1 * wy1))

    dbt = jax.lax.dot_general(imgf_ref[0], g_mat, (((1,), (1,)), ((), ())),
                              preferred_element_type=jnp.float32)
    scores_t = jax.lax.dot_general(dbt, q_ref[0], (((0,), (0,)), ((), ())),
                                   preferred_element_type=jnp.float32)
    for j in range(16):
        out_ref[0, j] = scores_t[:, j * _RPT:(j + 1) * _RPT]


def _sc_ap_kernel(scores_hbm, rel_hbm, out_hbm, buf, hnbs, hrec, relv, apqv):
    cid = lax.axis_index("c")
    sid = lax.axis_index("s")
    wid = sid * 2 + cid
    b = wid // 16
    j = wid % 16
    pltpu.sync_copy(rel_hbm.at[pl.ds(wid * _RPT, _RPT)], relv)

    zeros = jnp.zeros((16,), jnp.float32)

    def zero_loop(i, carry):
        hnbs[pl.ds(i * 16, 16)] = zeros
        hrec[pl.ds(i * 16, 16)] = zeros
        return carry
    lax.fori_loop(0, (_RPT * _NB) // 16, zero_loop, 0)

    lane = lax.iota(jnp.int32, 16)

    for ch in range(_HW // _CHUNK):
        pltpu.sync_copy(scores_hbm.at[b, j, pl.ds(ch * _CHUNK, _CHUNK)], buf)
        for sub in range(_RPT // 16):
            nl = sub * 16 + lane
            base = nl * _NB
            col = sub * 16

            def m_loop(m, carry):
                x = buf[m, pl.ds(col, 16)]
                t = jnp.clip(_A - _A * x, 0.0, _A)
                f = t.astype(jnp.int32)
                fr = t - f.astype(jnp.float32)
                plsc.addupdate_scatter(hnbs, [base + f], 1.0 - fr)
                plsc.addupdate_scatter(hnbs, [base + f + 1], fr)
                return carry
            lax.fori_loop(0, _CHUNK, m_loop, 0)

            nrow = j * _RPT + nl
            rr = nrow // _W
            cc = nrow % _W

            def pos_loop(i, carry):
                dr = i // 9 - 4
                dc = i % 9 - 4
                rv = rr + dr
                cv = cc + dc
                m = nrow + (dr * _W + dc)
                ok = ((rv >= 0) & (rv <= _H - 1) & (cv >= 0) & (cv <= _W - 1)
                      & (m >= ch * _CHUNK) & (m < (ch + 1) * _CHUNK))
                ml = jnp.clip(m - ch * _CHUNK, 0, _CHUNK - 1)
                x = plsc.load_gather(buf, [ml, nl], mask=ok)
                t = jnp.clip(_A - _A * x, 0.0, _A)
                f = t.astype(jnp.int32)
                fr = t - f.astype(jnp.float32)
                plsc.addupdate_scatter(hrec, [base + f], 1.0 - fr, mask=ok)
                plsc.addupdate_scatter(hrec, [base + f + 1], fr, mask=ok)
                return carry
            lax.fori_loop(0, 81, pos_loop, 0)

    for sub in range(_RPT // 16):
        nl = sub * 16 + lane
        base = nl * _NB
        col = sub * 16

        def k_loop(k, carry):
            cumn, prev, ap = carry
            nb = plsc.load_gather(hnbs, [base + k])
            rc = plsc.load_gather(hrec, [base + k])
            cumn = cumn + nb
            cumr = prev + rc
            ap = ap + cumr * (cumr - prev) / (1e-16 + cumn)
            return (cumn, cumr, ap)
        _, totr, ap = lax.fori_loop(0, _NQ, k_loop, (zeros, zeros, zeros))
        ap = ap / totr
        rv = relv[pl.ds(col, 16)]
        apqv[pl.ds(col, 16)] = 1.0 - (ap * rv + _K_COEF * (1.0 - rv))

    pltpu.sync_copy(apqv, out_hbm.at[pl.ds(wid * _RPT, _RPT)])


def kernel(image1_descriptor, image2_descriptor, reliability, grid):
    q = image1_descriptor.reshape(_B, _C, _HW)
    imgf = image2_descriptor.reshape(_B, _C, _HW)
    gridf = grid.reshape(_B, _HW, 2)
    relf = reliability.reshape(_B * _HW)

    scores_t = pl.pallas_call(
        _scores_kernel,
        grid=(_B,),
        in_specs=[
            pl.BlockSpec((1, _C, _HW), lambda i: (i, 0, 0)),
            pl.BlockSpec((1, _C, _HW), lambda i: (i, 0, 0)),
            pl.BlockSpec((1, _HW, 2), lambda i: (i, 0, 0)),
        ],
        out_specs=pl.BlockSpec((1, 16, _HW, _RPT), lambda i: (i, 0, 0, 0)),
        out_shape=jax.ShapeDtypeStruct((_B, 16, _HW, _RPT), jnp.float32),
    )(q, imgf, gridf)

    sc_fn = pl.kernel(
        _sc_ap_kernel,
        out_type=jax.ShapeDtypeStruct((_B * _HW,), jnp.float32),
        mesh=plsc.VectorSubcoreMesh(core_axis_name="c", subcore_axis_name="s"),
        compiler_params=pltpu.CompilerParams(needs_layout_passes=False),
        scratch_types=[
            pltpu.VMEM((_CHUNK, _RPT), jnp.float32),
            pltpu.VMEM((_RPT * _NB,), jnp.float32),
            pltpu.VMEM((_RPT * _NB,), jnp.float32),
            pltpu.VMEM((_RPT,), jnp.float32),
            pltpu.VMEM((_RPT,), jnp.float32),
        ],
    )
    apq = sc_fn(scores_t, relf)
    return jnp.sum(apq) / float(_B * _HW)


# speedup vs baseline: 2.9310x; 2.9310x over previous
import functools

import jax
import jax.numpy as jnp
from jax import lax
from jax.experimental import pallas as pl
from jax.experimental.pallas import tpu as pltpu
from jax.experimental.pallas import tpu_sc as plsc

_B, _C, _H, _W = 2, 128, 32, 32
_HW = _H * _W
_NQ = 25
_A = float(_NQ - 1)
_K_COEF = 0.5
_NTILES = 32
_RPT = (_B * _HW) // _NTILES
_NB = 32
_CHUNK = 256
_NBANK = 4


def _scores_kernel(q_ref, imgf_ref, grid_ref, out_ref):
    gx = grid_ref[0, :, 0:1]
    gy = grid_ref[0, :, 1:2]
    x = (gx + 1.0) * (_W / 2.0) - 0.5
    y = (gy + 1.0) * (_H / 2.0) - 0.5
    x0 = jnp.floor(x)
    y0 = jnp.floor(y)
    x1 = x0 + 1.0
    y1 = y0 + 1.0
    wx1 = x - x0
    wx0 = 1.0 - wx1
    wy1 = y - y0
    wy0 = 1.0 - wy1

    iota_p = jax.lax.broadcasted_iota(jnp.int32, (_HW, _HW), 1)

    def tap_mat(xi, yi, wgt):
        valid = (xi >= 0.0) & (xi <= _W - 1.0) & (yi >= 0.0) & (yi <= _H - 1.0)
        w = jnp.where(valid, wgt, 0.0)
        xc = jnp.clip(xi, 0.0, _W - 1.0).astype(jnp.int32)
        yc = jnp.clip(yi, 0.0, _H - 1.0).astype(jnp.int32)
        idx = yc * _W + xc
        return jnp.where(iota_p == idx, w, 0.0)

    g_mat = (tap_mat(x0, y0, wx0 * wy0) + tap_mat(x1, y0, wx1 * wy0)
             + tap_mat(x0, y1, wx0 * wy1) + tap_mat(x1, y1, wx1 * wy1))

    dbt = jax.lax.dot_general(imgf_ref[0], g_mat, (((1,), (1,)), ((), ())),
                              preferred_element_type=jnp.float32)
    scores_t = jax.lax.dot_general(dbt, q_ref[0], (((0,), (0,)), ((), ())),
                                   preferred_element_type=jnp.float32)
    for j in range(16):
        out_ref[0, j] = scores_t[:, j * _RPT:(j + 1) * _RPT]


def _sc_ap_kernel(scores_hbm, rel_hbm, out_hbm, buf0, buf1, hnbs, hrec,
                  relv, apqv, sem0, sem1):
    cid = lax.axis_index("c")
    sid = lax.axis_index("s")
    wid = sid * 2 + cid
    b = wid // 16
    j = wid % 16
    hsz = _RPT * _NB
    pltpu.sync_copy(rel_hbm.at[pl.ds(wid * _RPT, _RPT)], relv)

    zeros = jnp.zeros((16,), jnp.float32)

    @functools.partial(plsc.parallel_loop, 0, (_NBANK * hsz) // 16, unroll=8)
    def _zn(i):
        hnbs[pl.ds(i * 16, 16)] = zeros

    @functools.partial(plsc.parallel_loop, 0, (2 * hsz) // 16, unroll=8)
    def _zr(i):
        hrec[pl.ds(i * 16, 16)] = zeros

    lane = lax.iota(jnp.int32, 16)
    bases = [(sub * 16 + lane) * _NB for sub in range(_RPT // 16)]

    bufs = (buf0, buf1)
    sems = (sem0, sem1)
    nch = _HW // _CHUNK
    copies = [None] * nch
    copies[0] = pltpu.async_copy(
        scores_hbm.at[b, j, pl.ds(0, _CHUNK)], bufs[0], sems[0])
    for ch in range(nch):
        copies[ch].wait()
        if ch + 1 < nch:
            copies[ch + 1] = pltpu.async_copy(
                scores_hbm.at[b, j, pl.ds((ch + 1) * _CHUNK, _CHUNK)],
                bufs[(ch + 1) % 2], sems[(ch + 1) % 2])
        buf = bufs[ch % 2]

        @functools.partial(plsc.parallel_loop, 0, _CHUNK, unroll=_NBANK)
        def _dense(m):
            bank = (m % _NBANK) * hsz
            for sub in range(_RPT // 16):
                x = buf[m, pl.ds(sub * 16, 16)]
                t = jnp.clip(_A - _A * x, 0.0, _A)
                f = t.astype(jnp.int32)
                fr = t - f.astype(jnp.float32)
                idx = bank + bases[sub] + f
                plsc.addupdate_scatter(hnbs, [idx], 1.0 - fr)
                plsc.addupdate_scatter(hnbs, [idx + 1], fr)

        @functools.partial(plsc.parallel_loop, 0, 81, unroll=2)
        def _pos(i):
            dr = i // 9 - 4
            dc = i % 9 - 4
            bank = (i % 2) * hsz
            for sub in range(_RPT // 16):
                nl = sub * 16 + lane
                nrow = j * _RPT + nl
                rr = nrow // _W
                cc = nrow % _W
                rv = rr + dr
                cv = cc + dc
                m = nrow + (dr * _W + dc)
                ok = ((rv >= 0) & (rv <= _H - 1) & (cv >= 0) & (cv <= _W - 1)
                      & (m >= ch * _CHUNK) & (m < (ch + 1) * _CHUNK))
                ml = jnp.clip(m - ch * _CHUNK, 0, _CHUNK - 1)
                x = plsc.load_gather(buf, [ml, nl], mask=ok)
                t = jnp.clip(_A - _A * x, 0.0, _A)
                f = t.astype(jnp.int32)
                fr = t - f.astype(jnp.float32)
                idx = bank + bases[sub] + f
                plsc.addupdate_scatter(hrec, [idx], 1.0 - fr, mask=ok)
                plsc.addupdate_scatter(hrec, [idx + 1], fr, mask=ok)

    for sub in range(_RPT // 16):
        base = bases[sub]
        col = sub * 16

        def k_loop(k, carry):
            cumn, prev, ap = carry
            i0 = base + k
            nb = plsc.load_gather(hnbs, [i0])
            for bk in range(1, _NBANK):
                nb = nb + plsc.load_gather(hnbs, [i0 + bk * hsz])
            rc = plsc.load_gather(hrec, [i0]) + plsc.load_gather(hrec, [i0 + hsz])
            cumn = cumn + nb
            cumr = prev + rc
            ap = ap + cumr * (cumr - prev) / (1e-16 + cumn)
            return (cumn, cumr, ap)
        _, totr, ap = lax.fori_loop(0, _NQ, k_loop, (zeros, zeros, zeros))
        ap = ap / totr
        rv = relv[pl.ds(col, 16)]
        apqv[pl.ds(col, 16)] = 1.0 - (ap * rv + _K_COEF * (1.0 - rv))

    pltpu.sync_copy(apqv, out_hbm.at[pl.ds(wid * _RPT, _RPT)])


def kernel(image1_descriptor, image2_descriptor, reliability, grid):
    q = image1_descriptor.reshape(_B, _C, _HW)
    imgf = image2_descriptor.reshape(_B, _C, _HW)
    gridf = grid.reshape(_B, _HW, 2)
    relf = reliability.reshape(_B * _HW)

    scores_t = pl.pallas_call(
        _scores_kernel,
        grid=(_B,),
        in_specs=[
            pl.BlockSpec((1, _C, _HW), lambda i: (i, 0, 0)),
            pl.BlockSpec((1, _C, _HW), lambda i: (i, 0, 0)),
            pl.BlockSpec((1, _HW, 2), lambda i: (i, 0, 0)),
        ],
        out_specs=pl.BlockSpec((1, 16, _HW, _RPT), lambda i: (i, 0, 0, 0)),
        out_shape=jax.ShapeDtypeStruct((_B, 16, _HW, _RPT), jnp.float32),
    )(q, imgf, gridf)

    sc_fn = pl.kernel(
        _sc_ap_kernel,
        out_type=jax.ShapeDtypeStruct((_B * _HW,), jnp.float32),
        mesh=plsc.VectorSubcoreMesh(core_axis_name="c", subcore_axis_name="s"),
        compiler_params=pltpu.CompilerParams(needs_layout_passes=False),
        scratch_types=[
            pltpu.VMEM((_CHUNK, _RPT), jnp.float32),
            pltpu.VMEM((_CHUNK, _RPT), jnp.float32),
            pltpu.VMEM((_NBANK * _RPT * _NB,), jnp.float32),
            pltpu.VMEM((2 * _RPT * _NB,), jnp.float32),
            pltpu.VMEM((_RPT,), jnp.float32),
            pltpu.VMEM((_RPT,), jnp.float32),
            pltpu.SemaphoreType.DMA,
            pltpu.SemaphoreType.DMA,
        ],
    )
    apq = sc_fn(scores_t, relf)
    return jnp.sum(apq) / float(_B * _HW)
